# baseline (speedup 1.0000x reference)
import functools

import jax
import jax.numpy as jnp
from jax import lax
from jax.experimental import pallas as pl
from jax.experimental.pallas import tpu as pltpu
from jax.experimental.pallas import tpu_sc as plsc

VOCAB = 100000
NUM_SENSE = 3
EMB_DIM = 64
WIDE = 128

NUM_CORES = 2
NUM_SUBCORES = 16
NW = NUM_CORES * NUM_SUBCORES
LANES = 16
NVREG = EMB_DIM // LANES
NCHUNK = 8


def _sc_fused(emb2, dis2, idx, ctx):
    B = idx.shape[0]
    b_per_w = B // NW
    cb = b_per_w // NCHUNK
    n2 = 2 * cb
    mesh = plsc.VectorSubcoreMesh(core_axis_name="c", subcore_axis_name="s")

    rows_t = pltpu.VMEM((n2, WIDE), jnp.float32)
    idx2_t = pltpu.VMEM((n2,), jnp.int32)

    @functools.partial(
        pl.kernel,
        mesh=mesh,
        compiler_params=pltpu.CompilerParams(
            use_tc_tiling_on_sc=False, needs_layout_passes=False
        ),
        out_type=jax.ShapeDtypeStruct((B, EMB_DIM), jnp.float32),
        scratch_types=[
            pltpu.VMEM((b_per_w,), jnp.int32),
            idx2_t, idx2_t,
            rows_t, rows_t,
            rows_t, rows_t,
            pltpu.VMEM((cb, EMB_DIM), jnp.float32),
            pltpu.VMEM((cb, EMB_DIM), jnp.float32),
            pltpu.SemaphoreType.DMA, pltpu.SemaphoreType.DMA,
            pltpu.SemaphoreType.DMA, pltpu.SemaphoreType.DMA,
        ],
    )
    def k(emb_hbm, dis_hbm, idx_hbm, ctx_hbm, out_hbm,
          idx_v, idx2_a, idx2_b, er_a, er_b, dr_a, dr_b, ctx_v, out_v,
          sem_ea, sem_eb, sem_da, sem_db):
        wid = lax.axis_index("s") * NUM_CORES + lax.axis_index("c")
        base = wid * b_per_w
        pltpu.sync_copy(idx_hbm.at[pl.ds(base, b_per_w)], idx_v)

        bufs = ((idx2_a, er_a, dr_a, sem_ea, sem_da),
                (idx2_b, er_b, dr_b, sem_eb, sem_db))

        def issue(c):
            idx2, er, dr, se, sd = bufs[c % 2]

            @pl.loop(0, cb, step=LANES)
            def _(g):
                r0 = lax.shift_right_logical(
                    idx_v[pl.ds(c * cb + g, LANES)] * NUM_SENSE, 1)
                idx2[pl.ds(g, LANES)] = r0
                idx2[pl.ds(cb + g, LANES)] = r0 + 1

            ce = pltpu.async_copy(emb_hbm.at[idx2], er, se)
            cd = pltpu.async_copy(dis_hbm.at[idx2], dr, sd)
            return ce, cd

        def sense_vregs(ref, j, par, s, kk):
            off = 64 * par + 64 * s + LANES * kk
            row = j if off < WIDE else cb + j
            return ref[row, pl.ds(off % WIDE, LANES)]

        inflight = [None, None]
        inflight[0] = issue(0)
        for c in range(NCHUNK):
            if c + 1 < NCHUNK:
                inflight[(c + 1) % 2] = issue(c + 1)
            _, er, dr, _, _ = bufs[c % 2]
            ce, cd = inflight[c % 2]
            pltpu.sync_copy(ctx_hbm.at[pl.ds(base + c * cb, cb)], ctx_v)
            cd.wait()
            ce.wait()

            @pl.loop(0, cb)
            def _(j):
                lane = lax.rem(j, LANES)
                g = j - lane
                wv = idx_v[pl.ds(c * cb + g, LANES)]
                lanes16 = lax.iota(jnp.int32, LANES)
                par_s = jnp.sum(jnp.where(lanes16 == lane, wv, 0)) & 1

                for par in (0, 1):
                    @pl.when(par_s == par)
                    def _(j=j, par=par):
                            cv = [ctx_v[j, pl.ds(kk * LANES, LANES)]
                                  for kk in range(NVREG)]
                            ss = []
                            for s in range(NUM_SENSE):
                                acc = sense_vregs(dr, j, par, s, 0) * cv[0]
                                for kk in range(1, NVREG):
                                    acc += sense_vregs(dr, j, par, s, kk) * cv[kk]
                                ss.append(jnp.sum(acc))
                            m = jnp.maximum(ss[0], jnp.maximum(ss[1], ss[2]))
                            ev = [jnp.exp(lax.broadcast(ss[s] - m, (LANES,)))
                                  for s in range(NUM_SENSE)]
                            den = ev[0] + ev[1] + ev[2]
                            for kk in range(NVREG):
                                num = ev[0] * sense_vregs(er, j, par, 0, kk)
                                num += ev[1] * sense_vregs(er, j, par, 1, kk)
                                num += ev[2] * sense_vregs(er, j, par, 2, kk)
                                out_v[j, pl.ds(kk * LANES, LANES)] = num / den

            pltpu.sync_copy(out_v, out_hbm.at[pl.ds(base + c * cb, cb)])

    return k(emb2, dis2, idx, ctx)


def kernel(word_ids, ctx, emb_table, disamb_table):
    idx = word_ids.astype(jnp.int32)
    emb2 = emb_table.reshape(VOCAB * NUM_SENSE // 2, WIDE)
    dis2 = disamb_table.reshape(VOCAB * NUM_SENSE // 2, WIDE)
    return _sc_fused(emb2, dis2, idx, ctx)


# speedup vs baseline: 1.0206x; 1.0206x over previous
import functools

import jax
import jax.numpy as jnp
from jax import lax
from jax.experimental import pallas as pl
from jax.experimental.pallas import tpu as pltpu
from jax.experimental.pallas import tpu_sc as plsc

VOCAB = 100000
NUM_SENSE = 3
EMB_DIM = 64
WIDE = 128

NUM_CORES = 2
NUM_SUBCORES = 16
NW = NUM_CORES * NUM_SUBCORES
LANES = 16
NVREG = EMB_DIM // LANES
NCHUNK = 8


def _sc_fused(emb2, dis2, idx, ctx):
    B = idx.shape[0]
    b_per_w = B // NW
    cb = b_per_w // NCHUNK
    n2 = 2 * cb
    mesh = plsc.VectorSubcoreMesh(core_axis_name="c", subcore_axis_name="s")

    rows_t = pltpu.VMEM((n2, WIDE), jnp.float32)
    idx2_t = pltpu.VMEM((n2,), jnp.int32)

    @functools.partial(
        pl.kernel,
        mesh=mesh,
        compiler_params=pltpu.CompilerParams(
            use_tc_tiling_on_sc=True, needs_layout_passes=False
        ),
        out_type=jax.ShapeDtypeStruct((B, EMB_DIM), jnp.float32),
        scratch_types=[
            pltpu.VMEM((b_per_w,), jnp.int32),
            idx2_t, idx2_t,
            rows_t, rows_t,
            rows_t, rows_t,
            pltpu.VMEM((cb, EMB_DIM), jnp.float32),
            pltpu.VMEM((cb, EMB_DIM), jnp.float32),
            pltpu.SemaphoreType.DMA, pltpu.SemaphoreType.DMA,
            pltpu.SemaphoreType.DMA, pltpu.SemaphoreType.DMA,
        ],
    )
    def k(emb_hbm, dis_hbm, idx_hbm, ctx_hbm, out_hbm,
          idx_v, idx2_a, idx2_b, er_a, er_b, dr_a, dr_b, ctx_v, out_v,
          sem_ea, sem_eb, sem_da, sem_db):
        wid = lax.axis_index("s") * NUM_CORES + lax.axis_index("c")
        base = wid * b_per_w
        pltpu.sync_copy(idx_hbm.at[pl.ds(base, b_per_w)], idx_v)

        bufs = ((idx2_a, er_a, dr_a, sem_ea, sem_da),
                (idx2_b, er_b, dr_b, sem_eb, sem_db))

        def issue(c):
            idx2, er, dr, se, sd = bufs[c % 2]

            @pl.loop(0, cb, step=LANES)
            def _(g):
                r0 = lax.shift_right_logical(
                    idx_v[pl.ds(c * cb + g, LANES)] * NUM_SENSE, 1)
                idx2[pl.ds(g, LANES)] = r0
                idx2[pl.ds(cb + g, LANES)] = r0 + 1

            ce = pltpu.async_copy(emb_hbm.at[idx2], er, se)
            cd = pltpu.async_copy(dis_hbm.at[idx2], dr, sd)
            return ce, cd

        def sense_vregs(ref, j, par, s, kk):
            off = 64 * par + 64 * s + LANES * kk
            row = j if off < WIDE else cb + j
            return ref[row, pl.ds(off % WIDE, LANES)]

        inflight = [None, None]
        inflight[0] = issue(0)
        for c in range(NCHUNK):
            if c + 1 < NCHUNK:
                inflight[(c + 1) % 2] = issue(c + 1)
            _, er, dr, _, _ = bufs[c % 2]
            ce, cd = inflight[c % 2]
            pltpu.sync_copy(ctx_hbm.at[pl.ds(base + c * cb, cb)], ctx_v)
            cd.wait()
            ce.wait()

            @pl.loop(0, cb)
            def _(j):
                lane = lax.rem(j, LANES)
                g = j - lane
                wv = idx_v[pl.ds(c * cb + g, LANES)]
                lanes16 = lax.iota(jnp.int32, LANES)
                par_s = jnp.sum(jnp.where(lanes16 == lane, wv, 0)) & 1

                for par in (0, 1):
                    @pl.when(par_s == par)
                    def _(j=j, par=par):
                            cv = [ctx_v[j, pl.ds(kk * LANES, LANES)]
                                  for kk in range(NVREG)]
                            ss = []
                            for s in range(NUM_SENSE):
                                acc = sense_vregs(dr, j, par, s, 0) * cv[0]
                                for kk in range(1, NVREG):
                                    acc += sense_vregs(dr, j, par, s, kk) * cv[kk]
                                ss.append(jnp.sum(acc))
                            m = jnp.maximum(ss[0], jnp.maximum(ss[1], ss[2]))
                            ev = [jnp.exp(lax.broadcast(ss[s] - m, (LANES,)))
                                  for s in range(NUM_SENSE)]
                            den = ev[0] + ev[1] + ev[2]
                            for kk in range(NVREG):
                                num = ev[0] * sense_vregs(er, j, par, 0, kk)
                                num += ev[1] * sense_vregs(er, j, par, 1, kk)
                                num += ev[2] * sense_vregs(er, j, par, 2, kk)
                                out_v[j, pl.ds(kk * LANES, LANES)] = num / den

            pltpu.sync_copy(out_v, out_hbm.at[pl.ds(base + c * cb, cb)])

    return k(emb2, dis2, idx, ctx)


def kernel(word_ids, ctx, emb_table, disamb_table):
    idx = word_ids.astype(jnp.int32)
    emb2 = emb_table.reshape(VOCAB * NUM_SENSE // 2, WIDE)
    dis2 = disamb_table.reshape(VOCAB * NUM_SENSE // 2, WIDE)
    return _sc_fused(emb2, dis2, idx, ctx)


# speedup vs baseline: 1.1020x; 1.0798x over previous
import functools

import jax
import jax.numpy as jnp
from jax import lax
from jax.experimental import pallas as pl
from jax.experimental.pallas import tpu as pltpu
from jax.experimental.pallas import tpu_sc as plsc

VOCAB = 100000
NUM_SENSE = 3
EMB_DIM = 64
NROW = VOCAB * NUM_SENSE
WIN = 16

NUM_CORES = 2
NUM_SUBCORES = 16
NW = NUM_CORES * NUM_SUBCORES
LANES = 16
NVREG = EMB_DIM // LANES
CB = 8


def _sc_fused(emb_table, dis_table, idx, ctx):
    B = idx.shape[0]
    b_per_w = B // NW
    n_chunk = b_per_w // CB
    mesh = plsc.VectorSubcoreMesh(core_axis_name="c", subcore_axis_name="s")

    rows_t = pltpu.VMEM((CB * WIN, EMB_DIM), jnp.float32)

    @functools.partial(
        pl.kernel,
        mesh=mesh,
        compiler_params=pltpu.CompilerParams(
            use_tc_tiling_on_sc=True, needs_layout_passes=False
        ),
        out_type=jax.ShapeDtypeStruct((B, EMB_DIM), jnp.float32),
        scratch_types=[
            pltpu.VMEM((b_per_w,), jnp.int32),
            rows_t, rows_t,
            rows_t, rows_t,
            pltpu.VMEM((CB, EMB_DIM), jnp.float32),
            pltpu.VMEM((CB, EMB_DIM), jnp.float32),
            pltpu.SemaphoreType.DMA, pltpu.SemaphoreType.DMA,
            pltpu.SemaphoreType.DMA, pltpu.SemaphoreType.DMA,
        ],
    )
    def k(emb_hbm, dis_hbm, idx_hbm, ctx_hbm, out_hbm,
          idx_v, er_a, er_b, dr_a, dr_b, ctx_v, out_v,
          sem_ea, sem_eb, sem_da, sem_db):
        wid = lax.axis_index("s") * NUM_CORES + lax.axis_index("c")
        base = wid * b_per_w
        pltpu.sync_copy(idx_hbm.at[pl.ds(base, b_per_w)], idx_v)

        buf_a = (er_a, dr_a, sem_ea, sem_da)
        buf_b = (er_b, dr_b, sem_eb, sem_db)
        lanes16 = lax.iota(jnp.int32, LANES)

        def win_base(w):
            r3 = w * NUM_SENSE
            rb = jnp.minimum(r3 & ~7, NROW - WIN)
            return pl.multiple_of(rb, 8), r3

        def extract(wv, lane):
            return jnp.sum(jnp.where(lanes16 == lane, wv, 0))

        def issue(wbase, half, buf):
            er, dr, se, sd = buf
            wv = idx_v[pl.ds(wbase, LANES)]
            for kel in range(CB):
                w = extract(wv, half * CB + kel)
                rb, _ = win_base(w)
                dst = pl.ds(kel * WIN, WIN)
                pltpu.async_copy(emb_hbm.at[pl.ds(rb, WIN)], er.at[dst], se)
                pltpu.async_copy(dis_hbm.at[pl.ds(rb, WIN)], dr.at[dst], sd)

        def drain(buf):
            er, dr, se, sd = buf
            pltpu.make_async_copy(emb_hbm.at[pl.ds(0, CB * WIN)], er, se).wait()
            pltpu.make_async_copy(dis_hbm.at[pl.ds(0, CB * WIN)], dr, sd).wait()

        def compute(wbase, half, buf):
            er, dr, _, _ = buf
            wv = idx_v[pl.ds(wbase, LANES)]
            for kel in range(CB):
                w = extract(wv, half * CB + kel)
                rb, r3 = win_base(w)
                i0 = r3 - rb
                cv = [ctx_v[kel, pl.ds(kk * LANES, LANES)]
                      for kk in range(NVREG)]
                ss = []
                for s in range(NUM_SENSE):
                    row = kel * WIN + i0 + s
                    acc = dr[row, pl.ds(0, LANES)] * cv[0]
                    for kk in range(1, NVREG):
                        acc += dr[row, pl.ds(kk * LANES, LANES)] * cv[kk]
                    ss.append(jnp.sum(acc))
                m = jnp.maximum(ss[0], jnp.maximum(ss[1], ss[2]))
                ev = [jnp.exp(lax.broadcast(ss[s] - m, (LANES,)))
                      for s in range(NUM_SENSE)]
                den = ev[0] + ev[1] + ev[2]
                for kk in range(NVREG):
                    sl = pl.ds(kk * LANES, LANES)
                    num = ev[0] * er[kel * WIN + i0, sl]
                    num += ev[1] * er[kel * WIN + i0 + 1, sl]
                    num += ev[2] * er[kel * WIN + i0 + 2, sl]
                    out_v[kel, sl] = num / den

        def load_ctx(ci):
            pltpu.sync_copy(ctx_hbm.at[pl.ds(base + ci * CB, CB)], ctx_v)

        def flush_out(ci):
            pltpu.sync_copy(out_v, out_hbm.at[pl.ds(base + ci * CB, CB)])

        issue(0, 0, buf_a)

        @pl.loop(0, n_chunk, step=2)
        def _(ci):
            wbase = ci * CB
            issue(wbase, 1, buf_b)
            load_ctx(ci)
            drain(buf_a)
            compute(wbase, 0, buf_a)
            flush_out(ci)

            @pl.when(ci < n_chunk - 2)
            def _():
                issue(wbase + SCB_PAD, 0, buf_a)

            load_ctx(ci + 1)
            drain(buf_b)
            compute(wbase, 1, buf_b)
            flush_out(ci + 1)

    return k(emb_table, dis_table, idx, ctx)


SCB_PAD = 2 * CB


def kernel(word_ids, ctx, emb_table, disamb_table):
    idx = word_ids.astype(jnp.int32)
    return _sc_fused(emb_table, disamb_table, idx, ctx)
